# Initial kernel scaffold; baseline (speedup 1.0000x reference)
import jax
import jax.numpy as jnp
from jax.experimental import pallas as pl


def kernel(x_0, t, noise, sqrt_alpha_bar, sqrt_one_minus_alpha_bar):
    raise NotImplementedError("write your pallas kernel here")


# speedup vs baseline: 1.7203x; 1.7203x over previous
import functools

import jax
import jax.numpy as jnp
from jax import lax
from jax.experimental import pallas as pl
from jax.experimental.pallas import tpu as pltpu
from jax.experimental.pallas import tpu_sc as plsc

_L = 16
_NC, _NS = 2, 16
_NW = _NC * _NS
_CR = 2000
_CE = 3 * _CR


def _q_sample_sc(x0f, t, nzf, tab_ab, tab_mab, num_chunks):
  tlen = tab_ab.shape[0]
  mesh = plsc.VectorSubcoreMesh(
      core_axis_name="c", subcore_axis_name="s",
      num_cores=_NC, num_subcores=_NS)

  @functools.partial(
      pl.kernel,
      out_type=jax.ShapeDtypeStruct(x0f.shape, jnp.float32),
      mesh=mesh,
      compiler_params=pltpu.CompilerParams(needs_layout_passes=False),
      scratch_types=[
          pltpu.VMEM((tlen,), jnp.float32),
          pltpu.VMEM((tlen,), jnp.float32),
          pltpu.VMEM((_CE,), jnp.float32),
          pltpu.VMEM((_CE,), jnp.float32),
          pltpu.VMEM((_CE,), jnp.float32),
          pltpu.VMEM((_CR,), jnp.int32),
      ],
  )
  def k(x0_hbm, t_hbm, nz_hbm, ab_hbm, mab_hbm, out_hbm,
        ab_v, mab_v, x0_v, nz_v, out_v, t_v):
    wid = lax.axis_index("s") * _NC + lax.axis_index("c")
    pltpu.sync_copy(ab_hbm, ab_v)
    pltpu.sync_copy(mab_hbm, mab_v)
    lane = lax.iota(jnp.int32, _L)
    three = jnp.full((_L,), 3, jnp.int32)
    nk = (num_chunks - 1 - wid) // _NW + 1

    def chunk_body(kk, carry):
      cid = kk * _NW + wid
      eoff = cid * _CE
      roff = cid * _CR
      pltpu.sync_copy(x0_hbm.at[pl.ds(eoff, _CE)], x0_v)
      pltpu.sync_copy(nz_hbm.at[pl.ds(eoff, _CE)], nz_v)
      pltpu.sync_copy(t_hbm.at[pl.ds(roff, _CR)], t_v)

      def inner(i, c):
        le = i * _L + lane
        row = lax.div(le, three)
        tv = plsc.load_gather(t_v, [row])
        s_ab = plsc.load_gather(ab_v, [tv])
        s_mab = plsc.load_gather(mab_v, [tv])
        x0x = x0_v[pl.ds(i * _L, _L)]
        nzx = nz_v[pl.ds(i * _L, _L)]
        out_v[pl.ds(i * _L, _L)] = s_ab * x0x + s_mab * nzx
        return c

      lax.fori_loop(0, _CE // _L, inner, 0)
      pltpu.sync_copy(out_v, out_hbm.at[pl.ds(eoff, _CE)])
      return carry

    lax.fori_loop(0, nk, chunk_body, 0)

  return k(x0f, t, nzf, tab_ab, tab_mab)


def kernel(x_0, t, noise, sqrt_alpha_bar, sqrt_one_minus_alpha_bar):
  n = x_0.shape[0]
  assert n % _CR == 0
  out_flat = _q_sample_sc(
      x_0.reshape(-1),
      t.astype(jnp.int32),
      noise.reshape(-1),
      sqrt_alpha_bar,
      sqrt_one_minus_alpha_bar,
      n // _CR,
  )
  return out_flat.reshape(n, 3), noise
